# Initial kernel scaffold; baseline (speedup 1.0000x reference)
import jax
import jax.numpy as jnp
from jax.experimental import pallas as pl


def kernel(x):
    raise NotImplementedError("write your pallas kernel here")


# speedup vs baseline: 1.7625x; 1.7625x over previous
import functools

import jax
import jax.numpy as jnp
from jax import lax
from jax.experimental import pallas as pl
from jax.experimental.pallas import tpu as pltpu
from jax.experimental.pallas import tpu_sc as plsc

_P = 0.12
_NC = 2
_NS = 16
_NW = _NC * _NS
_CHUNK = 16


def _gather_rows(b, s):
    probs = jnp.array([_P / 2, 1.0 - _P, _P / 2], dtype=jnp.float32)
    logits = jnp.log(probs)
    k = jax.random.fold_in(jax.random.key(42), 1)
    off = jax.random.categorical(k, logits, shape=(b, s)) - 1
    off = off.at[:, 0].set(jnp.clip(off[:, 0], 0, 1))
    off = off.at[:, -1].set(jnp.clip(off[:, -1], -1, 0))
    rows = off + jnp.arange(s, dtype=off.dtype)[None, :]
    rows = rows + (jnp.arange(b, dtype=off.dtype) * s)[:, None]
    return rows.reshape(-1).astype(jnp.int32)


@functools.partial(jax.jit, static_argnums=(2, 3))
def _sc_gather(xf, idx, r, c):
    rows_per_w = r // _NW
    nsteps = rows_per_w // _CHUNK
    mesh = plsc.VectorSubcoreMesh(core_axis_name="c", subcore_axis_name="s")

    @functools.partial(
        pl.kernel,
        mesh=mesh,
        out_type=jax.ShapeDtypeStruct((r, c), jnp.float32),
        scratch_types=[
            pltpu.VMEM((nsteps, _CHUNK), jnp.int32),
            pltpu.VMEM((_CHUNK, c), jnp.float32),
            pltpu.VMEM((_CHUNK, c), jnp.float32),
            pltpu.SemaphoreType.DMA,
            pltpu.SemaphoreType.DMA,
            pltpu.SemaphoreType.DMA,
            pltpu.SemaphoreType.DMA,
        ],
    )
    def k(x_hbm, idx_hbm, out_hbm, idx_v, buf0, buf1, g0, g1, w0, w1):
        bufs = (buf0, buf1)
        gsem = (g0, g1)
        wsem = (w0, w1)
        wid = lax.axis_index("s") * _NC + lax.axis_index("c")
        base = wid * rows_per_w
        pltpu.sync_copy(idx_hbm.at[wid], idx_v)

        def gather(j, b):
            return pltpu.make_async_copy(x_hbm.at[idx_v.at[j]], bufs[b], gsem[b])

        def write(j, b):
            return pltpu.make_async_copy(
                bufs[b], out_hbm.at[pl.ds(base + j * _CHUNK, _CHUNK)], wsem[b]
            )

        gather(0, 0).start()
        gather(1, 1).start()

        def body(i, _):
            j = i * 2
            for b in range(2):
                jj = j + b
                gather(jj, b).wait()
                write(jj, b).start()

                @pl.when(jj + 2 < nsteps)
                def _():
                    write(jj, b).wait()
                    gather(jj + 2, b).start()

            return 0

        lax.fori_loop(0, nsteps // 2, body, 0)
        write(nsteps - 2, 0).wait()
        write(nsteps - 1, 1).wait()

    return k(xf, idx)


def kernel(x):
    b, s, c = x.shape
    r = b * s
    rows = _gather_rows(b, s).reshape(_NW, r // _NW // _CHUNK, _CHUNK)
    out = _sc_gather(x.reshape(r, c), rows, r, c)
    return out.reshape(b, s, c)
